# baseline (speedup 1.0000x reference)
import functools

import jax
import jax.numpy as jnp
from jax import lax
from jax.experimental import pallas as pl
from jax.experimental.pallas import tpu as pltpu
from jax.experimental.pallas import tpu_sc as plsc

ROWS_PER_CHUNK = 2


@functools.lru_cache(maxsize=None)
def _make_gather(B, S, D):
    info = plsc.get_sparse_core_info()
    NC, NS = info.num_cores, info.num_subcores
    NW = NC * NS
    assert B % (NW * ROWS_PER_CHUNK) == 0
    rows_per_w = B // NW
    n_chunks = rows_per_w // ROWS_PER_CHUNK
    assert n_chunks % 2 == 0 and n_chunks >= 4
    mesh = plsc.VectorSubcoreMesh(core_axis_name="c", subcore_axis_name="s")

    @functools.partial(
        pl.kernel,
        mesh=mesh,
        out_type=jax.ShapeDtypeStruct((B, S, D), jnp.float32),
        compiler_params=pltpu.CompilerParams(use_tc_tiling_on_sc=False),
        scratch_types=[
            pltpu.VMEM((rows_per_w, S), jnp.int32),
            pltpu.VMEM((ROWS_PER_CHUNK, S, D), jnp.float32),
            pltpu.VMEM((ROWS_PER_CHUNK, S, D), jnp.float32),
            pltpu.SemaphoreType.DMA,
            pltpu.SemaphoreType.DMA,
            pltpu.SemaphoreType.DMA,
            pltpu.SemaphoreType.DMA,
        ],
    )
    def gather_kernel(x_hbm, table_hbm, out_hbm, idx_v, buf0, buf1,
                      sg0, sg1, so0, so1):
        wid = lax.axis_index("s") * NC + lax.axis_index("c")
        base = wid * rows_per_w
        bufs = (buf0, buf1)
        sgs = (sg0, sg1)
        sos = (so0, so1)

        def start_gather(c, b):
            for j in range(ROWS_PER_CHUNK):
                pltpu.async_copy(
                    table_hbm.at[idx_v.at[c * ROWS_PER_CHUNK + j]],
                    bufs[b].at[j], sgs[b])

        def wait_gather(b):
            for j in range(ROWS_PER_CHUNK):
                pltpu.make_async_copy(
                    table_hbm.at[idx_v.at[j]], bufs[b].at[j], sgs[b]).wait()

        def start_out(c, b):
            pltpu.async_copy(
                bufs[b],
                out_hbm.at[pl.ds(base + c * ROWS_PER_CHUNK, ROWS_PER_CHUNK)],
                sos[b])

        def wait_out(c, b):
            pltpu.make_async_copy(
                bufs[b],
                out_hbm.at[pl.ds(base + c * ROWS_PER_CHUNK, ROWS_PER_CHUNK)],
                sos[b]).wait()

        pltpu.sync_copy(x_hbm.at[pl.ds(base, rows_per_w)], idx_v)

        start_gather(0, 0)
        wait_gather(0)
        start_out(0, 0)
        start_gather(1, 1)

        def body(i, carry):
            c = 1 + 2 * i
            for b, cc in ((1, c), (0, c + 1)):
                wait_gather(b)
                start_out(cc, b)
                wait_out(cc - 1, b ^ 1)
                start_gather(cc + 1, b ^ 1)
            return carry

        lax.fori_loop(0, (n_chunks - 2) // 2, body, 0)

        wait_gather(1)
        start_out(n_chunks - 1, 1)
        wait_out(n_chunks - 2, 0)
        wait_out(n_chunks - 1, 1)

    return gather_kernel


def _kernel_impl(x, word_emb):
    B, S = x.shape
    D = word_emb.shape[1]
    return _make_gather(B, S, D)(x.astype(jnp.int32), word_emb)


from jax.experimental.layout import Format, Layout

kernel = jax.jit(
    _kernel_impl,
    out_shardings=Format(
        Layout(major_to_minor=(0, 1, 2), tiling=()),
        jax.sharding.SingleDeviceSharding(jax.devices()[0]),
    ),
)
